# baseline (speedup 1.0000x reference)
import jax
import jax.numpy as jnp
from jax.experimental import pallas as pl
from jax.experimental.pallas import tpu as pltpu


def _pe_kernel(a_ref, w_ref, pos_ref, cls_ref, b_ref, out_ref):
    TB, Mp, K = a_ref.shape
    M = out_ref.shape[1]
    D = w_ref.shape[1]
    emb = jnp.dot(
        a_ref[...].reshape(TB * Mp, K), w_ref[...],
        preferred_element_type=jnp.float32,
    ).reshape(TB, Mp, D)
    pb = jnp.concatenate(
        [cls_ref[...] + pos_ref[0:1, :], pos_ref[1:, :] + b_ref[...]], axis=0)
    out_ref[...] = emb[:, :M, :] + pb[None]


def _vit_patch_embed(x, conv_w, conv_b, cls_token, pos_embed, patch_size,
                     *, batch_tile=16):
    B, C, H, W = x.shape
    ph, pw = patch_size
    gh, gw = H // ph, W // pw
    N = gh * gw
    D = conv_w.shape[0]
    K = C * ph * pw
    assert pos_embed.shape[1] == N + 1
    Mp = ((N + 1 + 7) // 8) * 8

    xc = x.astype(jnp.bfloat16)
    patches = xc.reshape(B, C, gh, ph, gw, pw).transpose(0, 2, 4, 1, 3, 5)
    patches = patches.reshape(B, N, K)
    patches = jnp.pad(patches, ((0, 0), (1, Mp - 1 - N), (0, 0)))

    w_mat = conv_w.reshape(D, K).T.astype(jnp.bfloat16)

    TB = batch_tile
    grid = (B // TB,)

    out = pl.pallas_call(
        _pe_kernel,
        out_shape=jax.ShapeDtypeStruct((B, N + 1, D), x.dtype),
        grid_spec=pltpu.PrefetchScalarGridSpec(
            num_scalar_prefetch=0,
            grid=grid,
            in_specs=[
                pl.BlockSpec((TB, Mp, K), lambda b: (b, 0, 0)),
                pl.BlockSpec((K, D), lambda b: (0, 0)),
                pl.BlockSpec((N + 1, D), lambda b: (0, 0)),
                pl.BlockSpec((1, D), lambda b: (0, 0)),
                pl.BlockSpec((1, D), lambda b: (0, 0)),
            ],
            out_specs=pl.BlockSpec((TB, N + 1, D), lambda b: (b, 0, 0)),
        ),
        compiler_params=pltpu.CompilerParams(
            dimension_semantics=("parallel",),
            vmem_limit_bytes=100 * 1024 * 1024,
        ),
    )(patches, w_mat, pos_embed[0], cls_token.reshape(1, D),
      conv_b.reshape(1, D))
    return out


def kernel(x, conv_w, conv_b, cls_token, pos_embed):
    B = x.shape[0]
    h = B // 2
    halves = [
        _vit_patch_embed(x[i * h:(i + 1) * h], conv_w, conv_b, cls_token,
                         pos_embed, (16, 16))
        for i in range(2)
    ]
    return jnp.concatenate(halves, axis=0)


# speedup vs baseline: 2.0227x; 2.0227x over previous
import jax
import jax.numpy as jnp
from jax.experimental import pallas as pl
from jax.experimental.pallas import tpu as pltpu


def _pe_kernel(a_ref, w_ref, pos_ref, cls_ref, b_ref, out_ref):
    Mp, TB, K = a_ref.shape
    M = out_ref.shape[0]
    D = w_ref.shape[1]
    emb = jnp.dot(
        a_ref[...].reshape(Mp * TB, K), w_ref[...],
        preferred_element_type=jnp.float32,
    ).reshape(Mp, TB, D)
    pb = jnp.concatenate(
        [cls_ref[...] + pos_ref[0:1, :], pos_ref[1:, :] + b_ref[...]], axis=0)
    out_ref[...] = emb[:M] + pb[:, None, :]


def _vit_patch_embed(x, conv_w, conv_b, cls_token, pos_embed, patch_size,
                     *, batch_tile=16):
    B, C, H, W = x.shape
    ph, pw = patch_size
    gh, gw = H // ph, W // pw
    N = gh * gw
    D = conv_w.shape[0]
    K = C * ph * pw
    assert pos_embed.shape[1] == N + 1
    Mp = ((N + 1 + 15) // 16) * 16

    xc = x.astype(jnp.bfloat16)
    patches = xc.reshape(B, C, gh, ph, gw, pw).transpose(2, 4, 0, 1, 3, 5)
    patches = patches.reshape(N, B, K)
    patches = jnp.pad(patches, ((1, Mp - 1 - N), (0, 0), (0, 0)))

    w_mat = conv_w.reshape(D, K).T.astype(jnp.bfloat16)

    TB = batch_tile
    grid = (B // TB,)

    out_t = pl.pallas_call(
        _pe_kernel,
        out_shape=jax.ShapeDtypeStruct((N + 1, B, D), x.dtype),
        grid_spec=pltpu.PrefetchScalarGridSpec(
            num_scalar_prefetch=0,
            grid=grid,
            in_specs=[
                pl.BlockSpec((Mp, TB, K), lambda b: (0, b, 0)),
                pl.BlockSpec((K, D), lambda b: (0, 0)),
                pl.BlockSpec((N + 1, D), lambda b: (0, 0)),
                pl.BlockSpec((1, D), lambda b: (0, 0)),
                pl.BlockSpec((1, D), lambda b: (0, 0)),
            ],
            out_specs=pl.BlockSpec((N + 1, TB, D), lambda b: (0, b, 0)),
        ),
        compiler_params=pltpu.CompilerParams(
            dimension_semantics=("parallel",),
            vmem_limit_bytes=100 * 1024 * 1024,
        ),
    )(patches, w_mat, pos_embed[0], cls_token.reshape(1, D),
      conv_b.reshape(1, D))
    return out_t.transpose(1, 0, 2)


def kernel(x, conv_w, conv_b, cls_token, pos_embed):
    return _vit_patch_embed(x, conv_w, conv_b, cls_token, pos_embed, (16, 16))
